# baseline (speedup 1.0000x reference)
import functools

import jax
import jax.numpy as jnp
from jax import lax
from jax.experimental import pallas as pl
from jax.experimental.pallas import tpu as pltpu
from jax.experimental.pallas import tpu_sc as plsc

B, N, C = 8, 20000, 91
L = 16
NC, NS = 2, 16
NW = NC * NS
WPI = NW // B
GPI = N // L
WG = -(-GPI // WPI)
K = 16
NCHUNK = -(-WG // K)
CHB = K * L
CHW = CHB * C
GSZ = L * C
OFFS = (0, 16, 32, 48, 64, 75)
BPAD = 8


def _permute(g, idx):
  dn = lax.GatherDimensionNumbers(offset_dims=(), collapsed_slice_dims=(0,),
                                  start_index_map=(0,))
  return lax.gather(g, idx[:, None], dn, (1,),
                    mode=lax.GatherScatterMode.PROMISE_IN_BOUNDS)


def _worker_body(lg_hbm, bx_hbm, pr_hbm, ts_hbm, sc_hbm, bo_hbm,
                 lg_v, bx_v, bo_v, sb_v, pr_v, a_v, ts_v, sem0, sem1):
  wid = lax.axis_index("s") * NC + lax.axis_index("c")
  img = wid // WPI
  q = wid % WPI
  g0 = jnp.minimum(q * WG, GPI - WG)
  box0 = img * N + g0 * L

  iota = lax.iota(jnp.int32, L)

  pltpu.sync_copy(pr_hbm.at[pl.ds(img * 96, 96)], pr_v)
  pltpu.sync_copy(ts_hbm.at[pl.ds(img * L, L)], ts_v)
  for j in range(96 // L):
    p = pr_v[pl.ds(j * L, L)]
    a_v[pl.ds(j * L, L)] = 1.0 + jnp.exp(-p)
  avecs = [a_v[pl.ds(o, L)] for o in OFFS]

  scalev = ts_v[...]
  hi = ((iota >> 1) & 1) == 1
  half = jnp.where(hi, 0.5, -0.5)
  perms = [iota ^ k for k in (8, 4, 2, 1)]

  def issue(t, buf_off, sem):
    cg = jnp.minimum(t * K, WG - K)
    src = (box0 + cg * L) * C
    pltpu.async_copy(lg_hbm.at[pl.ds(src, CHW)],
                     lg_v.at[pl.ds(buf_off, CHW)], sem)

  issue(0, 0, sem0)
  issue(1, CHW, sem1)

  def chunk(t, buf_off, sem):
    cg = jnp.minimum(t * K, WG - K)
    boxb = box0 + cg * L
    pltpu.sync_copy(bx_hbm.at[pl.ds(boxb * 4, CHB * 4)],
                    bx_v.at[pl.ds(BPAD, CHB * 4)])
    pltpu.make_async_copy(lg_hbm.at[pl.ds(0, CHW)],
                          lg_v.at[pl.ds(buf_off, CHW)], sem).wait()

    def grp_body(g, carry):
      gbase = buf_off + g * GSZ

      def box_body(i, acc):
        o = gbase + i * C
        t = None
        for j, off in enumerate(OFFS):
          x = lg_v[pl.ds(o + off, L)]
          v = avecs[j] * jnp.exp(-x) + avecs[j]
          t = v if t is None else jnp.minimum(t, v)
        for pm in perms:
          t = jnp.minimum(t, _permute(t, pm))
        return jnp.where(iota == jnp.broadcast_to(i, (L,)), t, acc)

      acc = lax.fori_loop(0, L, box_body, jnp.zeros((L,), jnp.float32),
                          unroll=4)
      sb_v[pl.ds(g * L, L)] = 1.0 / acc
      return carry

    lax.fori_loop(0, K, grp_body, 0)

    def bx_body(j, carry):
      o = BPAD + j * L
      v = bx_v[pl.ds(o, L)]
      vm2 = bx_v[pl.ds(o - 2, L)]
      vp2 = bx_v[pl.ds(o + 2, L)]
      cvec = jnp.where(hi, vm2, v)
      wvec = jnp.where(hi, v, vp2)
      bo_v[pl.ds(j * L, L)] = (cvec + half * wvec) * scalev
      return carry

    lax.fori_loop(0, CHB * 4 // L, bx_body, 0)

    pltpu.sync_copy(sb_v, sc_hbm.at[pl.ds(boxb, CHB)])
    pltpu.sync_copy(bo_v, bo_hbm.at[pl.ds(boxb * 4, CHB * 4)])

  def pair_body(i, carry):
    t0 = 2 * i
    chunk(t0, 0, sem0)

    @pl.when(t0 + 2 < NCHUNK)
    def _issue0():
      issue(t0 + 2, 0, sem0)

    chunk(t0 + 1, CHW, sem1)

    @pl.when(t0 + 3 < NCHUNK)
    def _issue1():
      issue(t0 + 3, CHW, sem1)

    return carry

  lax.fori_loop(0, NCHUNK // 2, pair_body, 0)


_sc_post = functools.partial(
    pl.kernel,
    out_type=(jax.ShapeDtypeStruct((B * N,), jnp.float32),
              jax.ShapeDtypeStruct((B * N * 4,), jnp.float32)),
    mesh=plsc.VectorSubcoreMesh(core_axis_name="c", subcore_axis_name="s",
                                num_cores=NC, num_subcores=NS),
    scratch_types=[
        pltpu.VMEM((2 * CHW,), jnp.float32),
        pltpu.VMEM((CHB * 4 + 2 * BPAD,), jnp.float32),
        pltpu.VMEM((CHB * 4,), jnp.float32),
        pltpu.VMEM((CHB,), jnp.float32),
        pltpu.VMEM((96,), jnp.float32),
        pltpu.VMEM((96,), jnp.float32),
        pltpu.VMEM((L,), jnp.float32),
        pltpu.SemaphoreType.DMA,
        pltpu.SemaphoreType.DMA,
    ])(_worker_body)


def kernel(pred_logits, pred_boxes, presence_logit_dec,
           target_sizes_boxes, target_sizes_masks):
  del target_sizes_masks
  lg = pred_logits.reshape(-1)
  bx = pred_boxes.reshape(-1)
  pr = jnp.pad(presence_logit_dec, ((0, 0), (0, 96 - C))).reshape(-1)
  wh = target_sizes_boxes[:, ::-1].astype(jnp.float32)
  ts = jnp.tile(wh, (1, L // 2)).reshape(-1)
  scores_f, boxes_f = _sc_post(lg, bx, pr, ts)
  scores = scores_f.reshape(B, N)
  labels = jnp.ones((B, N), jnp.int32)
  boxes = boxes_f.reshape(B, N, 4)
  return scores, labels, boxes
